# Initial kernel scaffold; baseline (speedup 1.0000x reference)
import functools

import jax
import jax.numpy as jnp
from jax import lax
from jax.experimental import pallas as pl
from jax.experimental.pallas import tpu as pltpu
from jax.experimental.pallas import tpu_sc as plsc

N = 10000
E = 320000
D_NODE = 128
D_EDGE = 16
H = 4

NUM_CORES = 2
NUM_SUBCORES = 16
NUM_TILES = NUM_CORES * NUM_SUBCORES
EDGES_PER_TILE = E // NUM_TILES
COPY_TILES = 10
ROWS_PER_COPY = N // COPY_TILES

_sc_mesh = plsc.VectorSubcoreMesh(core_axis_name="c", subcore_axis_name="s")


@functools.partial(
    pl.kernel,
    out_type=(
        jax.ShapeDtypeStruct((N, H), jnp.float32),
        jax.ShapeDtypeStruct((N, H), jnp.float32),
    ),
    mesh=_sc_mesh,
    scratch_types=[
        pltpu.VMEM((EDGES_PER_TILE,), jnp.int32),
        pltpu.VMEM((EDGES_PER_TILE, H), jnp.float32),
        pltpu.VMEM_SHARED((N, H), jnp.float32),
    ],
)
def _sc_scatter(dst_hbm, attn_hbm, zeros_hbm, out0_hbm, out1_hbm,
                idx_v, vals_v, acc_sh):
    c = lax.axis_index("c")
    s = lax.axis_index("s")
    wid = c * NUM_SUBCORES + s

    @pl.when(s < COPY_TILES)
    def _():
        sl = pl.ds(s * ROWS_PER_COPY, ROWS_PER_COPY)
        pltpu.sync_copy(zeros_hbm.at[sl], acc_sh.at[sl])

    base = wid * EDGES_PER_TILE
    pltpu.sync_copy(dst_hbm.at[pl.ds(base, EDGES_PER_TILE)], idx_v)
    pltpu.sync_copy(attn_hbm.at[pl.ds(base, EDGES_PER_TILE)], vals_v)

    plsc.subcore_barrier()
    pltpu.sync_copy(vals_v, acc_sh.at[idx_v], add=True)
    plsc.subcore_barrier()

    @pl.when(s < COPY_TILES)
    def _():
        sl = pl.ds(s * ROWS_PER_COPY, ROWS_PER_COPY)

        @pl.when(c == 0)
        def _():
            pltpu.sync_copy(acc_sh.at[sl], out0_hbm.at[sl])

        @pl.when(c == 1)
        def _():
            pltpu.sync_copy(acc_sh.at[sl], out1_hbm.at[sl])


def _node_mlp_body(p0_ref, p1_ref, nf_ref, wn1a_ref, wn1b_ref, bn1_ref,
                   wn2t_ref, bn2_ref, out_ref):
    nap = p0_ref[...] + p1_ref[...]
    hm = jnp.max(nap, axis=0, keepdims=True) + 1e-10
    napn = nap / hm
    h1 = nf_ref[...] @ wn1a_ref[...]
    for hh in range(H):
        h1 = h1 + napn[:, hh:hh + 1] * wn1b_ref[hh:hh + 1, :]
    h1 = h1 + bn1_ref[...]
    h1 = jax.nn.gelu(h1, approximate=False)
    logits = jnp.sum(h1 * wn2t_ref[...], axis=1, keepdims=True) + bn2_ref[...]
    out_ref[...] = jax.nn.sigmoid(logits)


_node_mlp = pl.pallas_call(
    _node_mlp_body,
    out_shape=jax.ShapeDtypeStruct((N, 1), jnp.float32),
)


_GATHER_ITERS = EDGES_PER_TILE // 16


@functools.partial(
    pl.kernel,
    out_type=(
        jax.ShapeDtypeStruct((E,), jnp.float32),
        jax.ShapeDtypeStruct((E,), jnp.float32),
    ),
    mesh=_sc_mesh,
    scratch_types=[
        pltpu.VMEM((N,), jnp.float32),
        pltpu.VMEM((EDGES_PER_TILE,), jnp.int32),
        pltpu.VMEM((EDGES_PER_TILE,), jnp.int32),
        pltpu.VMEM((EDGES_PER_TILE,), jnp.float32),
        pltpu.VMEM((EDGES_PER_TILE,), jnp.float32),
    ],
)
def _sc_gather(src_hbm, tgt_hbm, gates_hbm, outs_hbm, outt_hbm,
               gates_v, sidx_v, tidx_v, souts_v, soutt_v):
    c = lax.axis_index("c")
    s = lax.axis_index("s")
    wid = c * NUM_SUBCORES + s
    base = wid * EDGES_PER_TILE

    pltpu.sync_copy(gates_hbm, gates_v)
    pltpu.sync_copy(src_hbm.at[pl.ds(base, EDGES_PER_TILE)], sidx_v)
    pltpu.sync_copy(tgt_hbm.at[pl.ds(base, EDGES_PER_TILE)], tidx_v)

    def body(i, carry):
        sl = pl.ds(i * 16, 16)
        souts_v[sl] = plsc.load_gather(gates_v, [sidx_v[sl]])
        soutt_v[sl] = plsc.load_gather(gates_v, [tidx_v[sl]])
        return carry

    lax.fori_loop(0, _GATHER_ITERS, body, 0)

    pltpu.sync_copy(souts_v, outs_hbm.at[pl.ds(base, EDGES_PER_TILE)])
    pltpu.sync_copy(soutt_v, outt_hbm.at[pl.ds(base, EDGES_PER_TILE)])


EDGE_BLOCK = 8000
EDGE_GRID = E // EDGE_BLOCK


def _edge_mlp_body(ef_ref, attn_ref, sg_ref, tg_ref, we1a_ref, we1b_ref,
                   we1c_ref, be1_ref, we2t_ref, be2_ref, out_ref):
    h = ef_ref[...] @ we1a_ref[...]
    attn = attn_ref[...]
    for hh in range(H):
        h = h + attn[:, hh:hh + 1] * we1b_ref[hh:hh + 1, :]
    h = h + sg_ref[...] * we1c_ref[0:1, :]
    h = h + tg_ref[...] * we1c_ref[1:2, :]
    h = h + be1_ref[...]
    h = jax.nn.gelu(h, approximate=False)
    logits = jnp.sum(h * we2t_ref[...], axis=1, keepdims=True) + be2_ref[...]
    out_ref[...] = jax.nn.sigmoid(logits)


_edge_mlp = pl.pallas_call(
    _edge_mlp_body,
    grid=(EDGE_GRID,),
    in_specs=[
        pl.BlockSpec((EDGE_BLOCK, D_EDGE), lambda i: (i, 0)),
        pl.BlockSpec((EDGE_BLOCK, H), lambda i: (i, 0)),
        pl.BlockSpec((EDGE_BLOCK, 1), lambda i: (i, 0)),
        pl.BlockSpec((EDGE_BLOCK, 1), lambda i: (i, 0)),
        pl.BlockSpec((D_EDGE, D_EDGE), lambda i: (0, 0)),
        pl.BlockSpec((H, D_EDGE), lambda i: (0, 0)),
        pl.BlockSpec((2, D_EDGE), lambda i: (0, 0)),
        pl.BlockSpec((1, D_EDGE), lambda i: (0, 0)),
        pl.BlockSpec((1, D_EDGE), lambda i: (0, 0)),
        pl.BlockSpec((1, 1), lambda i: (0, 0)),
    ],
    out_specs=pl.BlockSpec((EDGE_BLOCK, 1), lambda i: (i, 0)),
    out_shape=jax.ShapeDtypeStruct((E, 1), jnp.float32),
)


def kernel(node_features, edge_features, edge_index, node_attn_weights,
           edge_attn_weights, Wn1, bn1, Wn2, bn2, We1, be1, We2, be2):
    src = edge_index[0]
    dst = edge_index[1]
    attn_h = node_attn_weights[:, :H]

    zeros = jnp.zeros((N, H), jnp.float32)
    p0, p1 = _sc_scatter(dst, attn_h, zeros)

    node_gates = _node_mlp(
        p0, p1, node_features,
        Wn1[:D_NODE], Wn1[D_NODE:],
        bn1.reshape(1, -1), Wn2.reshape(1, -1), bn2.reshape(1, 1),
    )

    src_g, tgt_g = _sc_gather(src, dst, node_gates[:, 0])

    edge_gates = _edge_mlp(
        edge_features, attn_h,
        src_g.reshape(E, 1), tgt_g.reshape(E, 1),
        We1[:D_EDGE], We1[D_EDGE:D_EDGE + H], We1[D_EDGE + H:],
        be1.reshape(1, -1), We2.reshape(1, -1), be2.reshape(1, 1),
    )

    return (node_gates[:, 0], edge_gates[:, 0])


# speedup vs baseline: 4.0632x; 4.0632x over previous
import functools

import jax
import jax.numpy as jnp
from jax import lax
from jax.experimental import pallas as pl
from jax.experimental.pallas import tpu as pltpu
from jax.experimental.pallas import tpu_sc as plsc

N = 10000
E = 320000
D_NODE = 128
D_EDGE = 16
H = 4

NUM_CORES = 2
NUM_SUBCORES = 16
NUM_TILES = NUM_CORES * NUM_SUBCORES
EDGES_PER_TILE = E // NUM_TILES
COPY_TILES = 10
ROWS_PER_COPY = N // COPY_TILES

_sc_mesh = plsc.VectorSubcoreMesh(core_axis_name="c", subcore_axis_name="s")
_sc_params = pltpu.CompilerParams(use_tc_tiling_on_sc=False,
                                  needs_layout_passes=False)

_INV_SQRT2 = 0.7071067811865476


def _gelu_exact(x):
    return x * 0.5 * (1.0 + lax.erf(x * _INV_SQRT2))


@functools.partial(
    pl.kernel,
    out_type=(
        jax.ShapeDtypeStruct((N, H), jnp.float32),
        jax.ShapeDtypeStruct((N, H), jnp.float32),
    ),
    mesh=_sc_mesh,
    compiler_params=_sc_params,
    scratch_types=[
        pltpu.VMEM((EDGES_PER_TILE,), jnp.int32),
        pltpu.VMEM((EDGES_PER_TILE, H), jnp.float32),
        pltpu.VMEM_SHARED((N, H), jnp.float32),
    ],
)
def _sc_scatter(dst_hbm, attn_hbm, zeros_hbm, out0_hbm, out1_hbm,
                idx_v, vals_v, acc_sh):
    c = lax.axis_index("c")
    s = lax.axis_index("s")
    wid = c * NUM_SUBCORES + s

    @pl.when(s < COPY_TILES)
    def _():
        sl = pl.ds(s * ROWS_PER_COPY, ROWS_PER_COPY)
        pltpu.sync_copy(zeros_hbm.at[sl], acc_sh.at[sl])

    base = wid * EDGES_PER_TILE
    pltpu.sync_copy(dst_hbm.at[pl.ds(base, EDGES_PER_TILE)], idx_v)
    pltpu.sync_copy(attn_hbm.at[pl.ds(base, EDGES_PER_TILE)], vals_v)

    plsc.subcore_barrier()
    pltpu.sync_copy(vals_v, acc_sh.at[idx_v], add=True)
    plsc.subcore_barrier()

    @pl.when(s < COPY_TILES)
    def _():
        sl = pl.ds(s * ROWS_PER_COPY, ROWS_PER_COPY)

        @pl.when(c == 0)
        def _():
            pltpu.sync_copy(acc_sh.at[sl], out0_hbm.at[sl])

        @pl.when(c == 1)
        def _():
            pltpu.sync_copy(acc_sh.at[sl], out1_hbm.at[sl])


def _node_mlp_body(p0_ref, p1_ref, nf_ref, wn1a_ref, wn1b_ref, bn1_ref,
                   wn2t_ref, bn2_ref, out_ref):
    nap = p0_ref[...] + p1_ref[...]
    hm = jnp.max(nap, axis=0, keepdims=True) + 1e-10
    napn = nap / hm
    h1 = nf_ref[...] @ wn1a_ref[...]
    for hh in range(H):
        h1 = h1 + napn[:, hh:hh + 1] * wn1b_ref[hh:hh + 1, :]
    h1 = h1 + bn1_ref[...]
    h1 = _gelu_exact(h1)
    logits = jnp.sum(h1 * wn2t_ref[...], axis=1, keepdims=True) + bn2_ref[...]
    out_ref[...] = jax.nn.sigmoid(logits)


_node_mlp = pl.pallas_call(
    _node_mlp_body,
    out_shape=jax.ShapeDtypeStruct((N, 1), jnp.float32),
)


_GATHER_ITERS = EDGES_PER_TILE // 16


@functools.partial(
    pl.kernel,
    out_type=(
        jax.ShapeDtypeStruct((E,), jnp.float32),
        jax.ShapeDtypeStruct((E,), jnp.float32),
    ),
    mesh=_sc_mesh,
    compiler_params=_sc_params,
    scratch_types=[
        pltpu.VMEM((N,), jnp.float32),
        pltpu.VMEM((EDGES_PER_TILE,), jnp.int32),
        pltpu.VMEM((EDGES_PER_TILE,), jnp.int32),
        pltpu.VMEM((EDGES_PER_TILE,), jnp.float32),
        pltpu.VMEM((EDGES_PER_TILE,), jnp.float32),
    ],
)
def _sc_gather(src_hbm, tgt_hbm, gates_hbm, outs_hbm, outt_hbm,
               gates_v, sidx_v, tidx_v, souts_v, soutt_v):
    c = lax.axis_index("c")
    s = lax.axis_index("s")
    wid = c * NUM_SUBCORES + s
    base = wid * EDGES_PER_TILE

    pltpu.sync_copy(gates_hbm, gates_v)
    pltpu.sync_copy(src_hbm.at[pl.ds(base, EDGES_PER_TILE)], sidx_v)
    pltpu.sync_copy(tgt_hbm.at[pl.ds(base, EDGES_PER_TILE)], tidx_v)

    def body(i, carry):
        sl = pl.ds(i * 16, 16)
        souts_v[sl] = plsc.load_gather(gates_v, [sidx_v[sl]])
        soutt_v[sl] = plsc.load_gather(gates_v, [tidx_v[sl]])
        return carry

    lax.fori_loop(0, _GATHER_ITERS, body, 0)

    pltpu.sync_copy(souts_v, outs_hbm.at[pl.ds(base, EDGES_PER_TILE)])
    pltpu.sync_copy(soutt_v, outt_hbm.at[pl.ds(base, EDGES_PER_TILE)])


EDGE_BLOCK = 8000
EDGE_GRID = E // EDGE_BLOCK


def _edge_mlp_body(ef_ref, attn_ref, sg_ref, tg_ref, we1a_ref, we1b_ref,
                   we1c_ref, be1_ref, we2t_ref, be2_ref, out_ref):
    h = ef_ref[...] @ we1a_ref[...]
    attn = attn_ref[...]
    for hh in range(H):
        h = h + attn[:, hh:hh + 1] * we1b_ref[hh:hh + 1, :]
    h = h + sg_ref[...] * we1c_ref[0:1, :]
    h = h + tg_ref[...] * we1c_ref[1:2, :]
    h = h + be1_ref[...]
    h = _gelu_exact(h)
    logits = jnp.sum(h * we2t_ref[...], axis=1, keepdims=True) + be2_ref[...]
    out_ref[...] = jax.nn.sigmoid(logits)


_edge_mlp = pl.pallas_call(
    _edge_mlp_body,
    grid=(EDGE_GRID,),
    in_specs=[
        pl.BlockSpec((EDGE_BLOCK, D_EDGE), lambda i: (i, 0)),
        pl.BlockSpec((EDGE_BLOCK, H), lambda i: (i, 0)),
        pl.BlockSpec((EDGE_BLOCK, 1), lambda i: (i, 0)),
        pl.BlockSpec((EDGE_BLOCK, 1), lambda i: (i, 0)),
        pl.BlockSpec((D_EDGE, D_EDGE), lambda i: (0, 0)),
        pl.BlockSpec((H, D_EDGE), lambda i: (0, 0)),
        pl.BlockSpec((2, D_EDGE), lambda i: (0, 0)),
        pl.BlockSpec((1, D_EDGE), lambda i: (0, 0)),
        pl.BlockSpec((1, D_EDGE), lambda i: (0, 0)),
        pl.BlockSpec((1, 1), lambda i: (0, 0)),
    ],
    out_specs=pl.BlockSpec((EDGE_BLOCK, 1), lambda i: (i, 0)),
    out_shape=jax.ShapeDtypeStruct((E, 1), jnp.float32),
)


def kernel(node_features, edge_features, edge_index, node_attn_weights,
           edge_attn_weights, Wn1, bn1, Wn2, bn2, We1, be1, We2, be2):
    src = edge_index[0]
    dst = edge_index[1]
    attn_h = node_attn_weights[:, :H]

    zeros = jnp.zeros((N, H), jnp.float32)
    p0, p1 = _sc_scatter(dst, attn_h, zeros)

    node_gates = _node_mlp(
        p0, p1, node_features,
        Wn1[:D_NODE], Wn1[D_NODE:],
        bn1.reshape(1, -1), Wn2.reshape(1, -1), bn2.reshape(1, 1),
    )

    src_g, tgt_g = _sc_gather(src, dst, node_gates[:, 0])

    edge_gates = _edge_mlp(
        edge_features, attn_h,
        src_g.reshape(E, 1), tgt_g.reshape(E, 1),
        We1[:D_EDGE], We1[D_EDGE:D_EDGE + H], We1[D_EDGE + H:],
        be1.reshape(1, -1), We2.reshape(1, -1), be2.reshape(1, 1),
    )

    return (node_gates[:, 0], edge_gates[:, 0])


# speedup vs baseline: 15.3279x; 3.7724x over previous
import functools

import jax
import jax.numpy as jnp
from jax import lax
from jax.experimental import pallas as pl
from jax.experimental.pallas import tpu as pltpu
from jax.experimental.pallas import tpu_sc as plsc

N = 10000
E = 320000
D_NODE = 128
D_EDGE = 16
H = 4

NUM_CORES = 2
NUM_SUBCORES = 16
NUM_TILES = NUM_CORES * NUM_SUBCORES
EDGES_PER_TILE = E // NUM_TILES
COPY_TILES = 10
WORDS_PER_COPY = N * H // COPY_TILES

_sc_mesh = plsc.VectorSubcoreMesh(core_axis_name="c", subcore_axis_name="s")
_sc_params = pltpu.CompilerParams(use_tc_tiling_on_sc=False,
                                  needs_layout_passes=False)

_INV_SQRT2 = 0.7071067811865476


def _gelu_exact(x):
    return x * 0.5 * (1.0 + lax.erf(x * _INV_SQRT2))


_IDX_ITERS = EDGES_PER_TILE // 16


@functools.partial(
    pl.kernel,
    out_type=(
        jax.ShapeDtypeStruct((N * H,), jnp.float32),
        jax.ShapeDtypeStruct((N * H,), jnp.float32),
    ),
    mesh=_sc_mesh,
    compiler_params=_sc_params,
    scratch_types=[
        pltpu.VMEM((EDGES_PER_TILE,), jnp.int32),
        pltpu.VMEM((EDGES_PER_TILE,), jnp.int32),
        pltpu.VMEM((EDGES_PER_TILE,), jnp.float32),
        pltpu.VMEM_SHARED((N * H,), jnp.float32),
    ],
)
def _sc_scatter(dst_hbm, attnt_hbm, zeros_hbm, out0_hbm, out1_hbm,
                idx_v, idx4_v, vals_v, acc_sh):
    c = lax.axis_index("c")
    s = lax.axis_index("s")
    wid = c * NUM_SUBCORES + s

    @pl.when(s < COPY_TILES)
    def _():
        sl = pl.ds(s * WORDS_PER_COPY, WORDS_PER_COPY)
        pltpu.sync_copy(zeros_hbm.at[sl], acc_sh.at[sl])

    base = wid * EDGES_PER_TILE
    pltpu.sync_copy(dst_hbm.at[pl.ds(base, EDGES_PER_TILE)], idx_v)

    plsc.subcore_barrier()

    for h in range(H):
        pltpu.sync_copy(attnt_hbm.at[h].at[pl.ds(base, EDGES_PER_TILE)],
                        vals_v)

        def body(i, carry, h=h):
            sl = pl.ds(i * 16, 16)
            idx4_v[sl] = idx_v[sl] * H + h
            return carry

        lax.fori_loop(0, _IDX_ITERS, body, 0)
        pltpu.sync_copy(vals_v, acc_sh.at[idx4_v], add=True)

    plsc.subcore_barrier()

    @pl.when(s < COPY_TILES)
    def _():
        sl = pl.ds(s * WORDS_PER_COPY, WORDS_PER_COPY)

        @pl.when(c == 0)
        def _():
            pltpu.sync_copy(acc_sh.at[sl], out0_hbm.at[sl])

        @pl.when(c == 1)
        def _():
            pltpu.sync_copy(acc_sh.at[sl], out1_hbm.at[sl])


def _node_mlp_body(p0_ref, p1_ref, nf_ref, wn1a_ref, wn1b_ref, bn1_ref,
                   wn2t_ref, bn2_ref, out_ref):
    nap = p0_ref[...] + p1_ref[...]
    hm = jnp.max(nap, axis=0, keepdims=True) + 1e-10
    napn = nap / hm
    h1 = nf_ref[...] @ wn1a_ref[...]
    for hh in range(H):
        h1 = h1 + napn[:, hh:hh + 1] * wn1b_ref[hh:hh + 1, :]
    h1 = h1 + bn1_ref[...]
    h1 = _gelu_exact(h1)
    logits = jnp.sum(h1 * wn2t_ref[...], axis=1, keepdims=True) + bn2_ref[...]
    out_ref[...] = jax.nn.sigmoid(logits)


_node_mlp = pl.pallas_call(
    _node_mlp_body,
    out_shape=jax.ShapeDtypeStruct((N, 1), jnp.float32),
)


_GATHER_ITERS = EDGES_PER_TILE // 16


@functools.partial(
    pl.kernel,
    out_type=(
        jax.ShapeDtypeStruct((E,), jnp.float32),
        jax.ShapeDtypeStruct((E,), jnp.float32),
    ),
    mesh=_sc_mesh,
    compiler_params=_sc_params,
    scratch_types=[
        pltpu.VMEM((N,), jnp.float32),
        pltpu.VMEM((EDGES_PER_TILE,), jnp.int32),
        pltpu.VMEM((EDGES_PER_TILE,), jnp.int32),
        pltpu.VMEM((EDGES_PER_TILE,), jnp.float32),
        pltpu.VMEM((EDGES_PER_TILE,), jnp.float32),
    ],
)
def _sc_gather(src_hbm, tgt_hbm, gates_hbm, outs_hbm, outt_hbm,
               gates_v, sidx_v, tidx_v, souts_v, soutt_v):
    c = lax.axis_index("c")
    s = lax.axis_index("s")
    wid = c * NUM_SUBCORES + s
    base = wid * EDGES_PER_TILE

    pltpu.sync_copy(gates_hbm, gates_v)
    pltpu.sync_copy(src_hbm.at[pl.ds(base, EDGES_PER_TILE)], sidx_v)
    pltpu.sync_copy(tgt_hbm.at[pl.ds(base, EDGES_PER_TILE)], tidx_v)

    def body(i, carry):
        sl = pl.ds(i * 16, 16)
        souts_v[sl] = plsc.load_gather(gates_v, [sidx_v[sl]])
        soutt_v[sl] = plsc.load_gather(gates_v, [tidx_v[sl]])
        return carry

    lax.fori_loop(0, _GATHER_ITERS, body, 0)

    pltpu.sync_copy(souts_v, outs_hbm.at[pl.ds(base, EDGES_PER_TILE)])
    pltpu.sync_copy(soutt_v, outt_hbm.at[pl.ds(base, EDGES_PER_TILE)])


EDGE_BLOCK = 6400
EDGE_GRID = E // EDGE_BLOCK


def _edge_mlp_body(ef_ref, attnt_ref, sg_ref, tg_ref, we1a_ref, we1bt_ref,
                   we1ct_ref, be1t_ref, we2_ref, be2_ref, out_ref):
    ht = lax.dot_general(we1a_ref[...], ef_ref[...],
                         (((0,), (1,)), ((), ())))
    attnt = attnt_ref[...]
    for hh in range(H):
        ht = ht + we1bt_ref[:, hh:hh + 1] * attnt[hh:hh + 1, :]
    ht = ht + we1ct_ref[:, 0:1] * sg_ref[...]
    ht = ht + we1ct_ref[:, 1:2] * tg_ref[...]
    ht = ht + be1t_ref[...]
    ht = _gelu_exact(ht)
    logits = jnp.sum(ht * we2_ref[...], axis=0, keepdims=True) + be2_ref[...]
    out_ref[...] = jax.nn.sigmoid(logits)


_edge_mlp = pl.pallas_call(
    _edge_mlp_body,
    grid=(EDGE_GRID,),
    in_specs=[
        pl.BlockSpec((EDGE_BLOCK, D_EDGE), lambda i: (i, 0)),
        pl.BlockSpec((H, EDGE_BLOCK), lambda i: (0, i)),
        pl.BlockSpec((1, EDGE_BLOCK), lambda i: (0, i)),
        pl.BlockSpec((1, EDGE_BLOCK), lambda i: (0, i)),
        pl.BlockSpec((D_EDGE, D_EDGE), lambda i: (0, 0)),
        pl.BlockSpec((D_EDGE, H), lambda i: (0, 0)),
        pl.BlockSpec((D_EDGE, 2), lambda i: (0, 0)),
        pl.BlockSpec((D_EDGE, 1), lambda i: (0, 0)),
        pl.BlockSpec((D_EDGE, 1), lambda i: (0, 0)),
        pl.BlockSpec((1, 1), lambda i: (0, 0)),
    ],
    out_specs=pl.BlockSpec((1, EDGE_BLOCK), lambda i: (0, i)),
    out_shape=jax.ShapeDtypeStruct((1, E), jnp.float32),
)


def kernel(node_features, edge_features, edge_index, node_attn_weights,
           edge_attn_weights, Wn1, bn1, Wn2, bn2, We1, be1, We2, be2):
    src = edge_index[0]
    dst = edge_index[1]
    attn_t = node_attn_weights.T

    zeros = jnp.zeros((N * H,), jnp.float32)
    p0, p1 = _sc_scatter(dst, attn_t, zeros)

    node_gates = _node_mlp(
        p0.reshape(N, H), p1.reshape(N, H), node_features,
        Wn1[:D_NODE], Wn1[D_NODE:],
        bn1.reshape(1, -1), Wn2.reshape(1, -1), bn2.reshape(1, 1),
    )

    src_g, tgt_g = _sc_gather(src, dst, node_gates[:, 0])

    edge_gates = _edge_mlp(
        edge_features, attn_t,
        src_g.reshape(1, E), tgt_g.reshape(1, E),
        We1[:D_EDGE], We1[D_EDGE:D_EDGE + H].T, We1[D_EDGE + H:].T,
        be1.reshape(-1, 1), We2, be2.reshape(1, 1),
    )

    return (node_gates[:, 0], edge_gates[0])


# speedup vs baseline: 23.5914x; 1.5391x over previous
import functools

import jax
import jax.numpy as jnp
from jax import lax
from jax.experimental import pallas as pl
from jax.experimental.pallas import tpu as pltpu
from jax.experimental.pallas import tpu_sc as plsc

N = 10000
E = 320000
D_NODE = 128
D_EDGE = 16
H = 4

NUM_CORES = 2
NUM_SUBCORES = 16
NUM_TILES = NUM_CORES * NUM_SUBCORES
EDGES_PER_TILE = E // NUM_TILES
COPY_TILES = 10
WORDS_PER_COPY = N * H // COPY_TILES

_sc_mesh = plsc.VectorSubcoreMesh(core_axis_name="c", subcore_axis_name="s")
_sc_params = pltpu.CompilerParams(use_tc_tiling_on_sc=False,
                                  needs_layout_passes=False)

_INV_SQRT2 = 0.7071067811865476


def _gelu_exact(x):
    return x * 0.5 * (1.0 + lax.erf(x * _INV_SQRT2))


_IDX_ITERS = EDGES_PER_TILE // 16


@functools.partial(
    pl.kernel,
    out_type=(
        jax.ShapeDtypeStruct((N * H,), jnp.float32),
        jax.ShapeDtypeStruct((N * H,), jnp.float32),
    ),
    mesh=_sc_mesh,
    compiler_params=_sc_params,
    scratch_types=[
        pltpu.VMEM((EDGES_PER_TILE,), jnp.int32),
        pltpu.VMEM((EDGES_PER_TILE,), jnp.int32),
        pltpu.VMEM((EDGES_PER_TILE,), jnp.float32),
        pltpu.VMEM_SHARED((N * H,), jnp.float32),
    ],
)
def _sc_scatter(dst_hbm, attnt_hbm, zeros_hbm, out0_hbm, out1_hbm,
                idx_v, idx4_v, vals_v, acc_sh):
    c = lax.axis_index("c")
    s = lax.axis_index("s")
    wid = c * NUM_SUBCORES + s

    @pl.when(s < COPY_TILES)
    def _():
        sl = pl.ds(s * WORDS_PER_COPY, WORDS_PER_COPY)
        pltpu.sync_copy(zeros_hbm.at[sl], acc_sh.at[sl])

    base = wid * EDGES_PER_TILE
    pltpu.sync_copy(dst_hbm.at[pl.ds(base, EDGES_PER_TILE)], idx_v)

    plsc.subcore_barrier()

    for h in range(H):
        pltpu.sync_copy(attnt_hbm.at[h].at[pl.ds(base, EDGES_PER_TILE)],
                        vals_v)

        def body(i, carry, h=h):
            sl = pl.ds(i * 16, 16)
            idx4_v[sl] = idx_v[sl] * H + h
            return carry

        lax.fori_loop(0, _IDX_ITERS, body, 0)
        pltpu.sync_copy(vals_v, acc_sh.at[idx4_v], add=True)

    plsc.subcore_barrier()

    @pl.when(s < COPY_TILES)
    def _():
        sl = pl.ds(s * WORDS_PER_COPY, WORDS_PER_COPY)

        @pl.when(c == 0)
        def _():
            pltpu.sync_copy(acc_sh.at[sl], out0_hbm.at[sl])

        @pl.when(c == 1)
        def _():
            pltpu.sync_copy(acc_sh.at[sl], out1_hbm.at[sl])


def _node_mlp_body(p0_ref, p1_ref, nf_ref, wn1a_ref, wn1b_ref, bn1_ref,
                   wn2t_ref, bn2_ref, out_ref):
    nap = p0_ref[...] + p1_ref[...]
    hm = jnp.max(nap, axis=0, keepdims=True) + 1e-10
    napn = nap / hm
    h1 = nf_ref[...] @ wn1a_ref[...]
    for hh in range(H):
        h1 = h1 + napn[:, hh:hh + 1] * wn1b_ref[hh:hh + 1, :]
    h1 = h1 + bn1_ref[...]
    h1 = _gelu_exact(h1)
    logits = jnp.sum(h1 * wn2t_ref[...], axis=1, keepdims=True) + bn2_ref[...]
    out_ref[...] = jax.nn.sigmoid(logits)


_node_mlp = pl.pallas_call(
    _node_mlp_body,
    out_shape=jax.ShapeDtypeStruct((N, 1), jnp.float32),
)


_GATHER_ITERS = EDGES_PER_TILE // 16


@functools.partial(
    pl.kernel,
    out_type=(
        jax.ShapeDtypeStruct((E,), jnp.float32),
        jax.ShapeDtypeStruct((E,), jnp.float32),
    ),
    mesh=_sc_mesh,
    compiler_params=_sc_params,
    scratch_types=[
        pltpu.VMEM((N,), jnp.float32),
        pltpu.VMEM((EDGES_PER_TILE,), jnp.int32),
        pltpu.VMEM((EDGES_PER_TILE,), jnp.int32),
        pltpu.VMEM((EDGES_PER_TILE,), jnp.float32),
        pltpu.VMEM((EDGES_PER_TILE,), jnp.float32),
    ],
)
def _sc_gather(src_hbm, tgt_hbm, gates_hbm, outs_hbm, outt_hbm,
               gates_v, sidx_v, tidx_v, souts_v, soutt_v):
    c = lax.axis_index("c")
    s = lax.axis_index("s")
    wid = c * NUM_SUBCORES + s
    base = wid * EDGES_PER_TILE

    pltpu.sync_copy(gates_hbm, gates_v)
    pltpu.sync_copy(src_hbm.at[pl.ds(base, EDGES_PER_TILE)], sidx_v)
    pltpu.sync_copy(tgt_hbm.at[pl.ds(base, EDGES_PER_TILE)], tidx_v)

    def body(i, carry):
        sl = pl.ds(i * 16, 16)
        souts_v[sl] = plsc.load_gather(gates_v, [sidx_v[sl]])
        soutt_v[sl] = plsc.load_gather(gates_v, [tidx_v[sl]])
        return carry

    lax.fori_loop(0, _GATHER_ITERS, body, 0)

    pltpu.sync_copy(souts_v, outs_hbm.at[pl.ds(base, EDGES_PER_TILE)])
    pltpu.sync_copy(soutt_v, outt_hbm.at[pl.ds(base, EDGES_PER_TILE)])


EDGE_BLOCK = 6400
EDGE_GRID = E // EDGE_BLOCK


def _edge_mlp_body(eft_ref, attnt_ref, sg_ref, tg_ref, we1a_ref, we1bt_ref,
                   we1ct_ref, be1t_ref, we2_ref, be2_ref, out_ref):
    ht = lax.dot_general(we1a_ref[...], eft_ref[...],
                         (((0,), (0,)), ((), ())))
    attnt = attnt_ref[...]
    for hh in range(H):
        ht = ht + we1bt_ref[:, hh:hh + 1] * attnt[hh:hh + 1, :]
    ht = ht + we1ct_ref[:, 0:1] * sg_ref[...]
    ht = ht + we1ct_ref[:, 1:2] * tg_ref[...]
    ht = ht + be1t_ref[...]
    ht = _gelu_exact(ht)
    logits = jnp.sum(ht * we2_ref[...], axis=0, keepdims=True) + be2_ref[...]
    out_ref[...] = jax.nn.sigmoid(logits)


_edge_mlp = pl.pallas_call(
    _edge_mlp_body,
    grid=(EDGE_GRID,),
    in_specs=[
        pl.BlockSpec((D_EDGE, EDGE_BLOCK), lambda i: (0, i)),
        pl.BlockSpec((H, EDGE_BLOCK), lambda i: (0, i)),
        pl.BlockSpec((1, EDGE_BLOCK), lambda i: (0, i)),
        pl.BlockSpec((1, EDGE_BLOCK), lambda i: (0, i)),
        pl.BlockSpec((D_EDGE, D_EDGE), lambda i: (0, 0)),
        pl.BlockSpec((D_EDGE, H), lambda i: (0, 0)),
        pl.BlockSpec((D_EDGE, 2), lambda i: (0, 0)),
        pl.BlockSpec((D_EDGE, 1), lambda i: (0, 0)),
        pl.BlockSpec((D_EDGE, 1), lambda i: (0, 0)),
        pl.BlockSpec((1, 1), lambda i: (0, 0)),
    ],
    out_specs=pl.BlockSpec((1, EDGE_BLOCK), lambda i: (0, i)),
    out_shape=jax.ShapeDtypeStruct((1, E), jnp.float32),
)


def kernel(node_features, edge_features, edge_index, node_attn_weights,
           edge_attn_weights, Wn1, bn1, Wn2, bn2, We1, be1, We2, be2):
    src = edge_index[0]
    dst = edge_index[1]
    attn_t = node_attn_weights.T

    zeros = jnp.zeros((N * H,), jnp.float32)
    p0, p1 = _sc_scatter(dst, attn_t, zeros)

    node_gates = _node_mlp(
        p0.reshape(N, H), p1.reshape(N, H), node_features,
        Wn1[:D_NODE], Wn1[D_NODE:],
        bn1.reshape(1, -1), Wn2.reshape(1, -1), bn2.reshape(1, 1),
    )

    src_g, tgt_g = _sc_gather(src, dst, node_gates[:, 0])

    edge_gates = _edge_mlp(
        edge_features.T, attn_t,
        src_g.reshape(1, E), tgt_g.reshape(1, E),
        We1[:D_EDGE], We1[D_EDGE:D_EDGE + H].T, We1[D_EDGE + H:].T,
        be1.reshape(-1, 1), We2, be2.reshape(1, 1),
    )

    return (node_gates[:, 0], edge_gates[0])


# speedup vs baseline: 38.7947x; 1.6444x over previous
import functools

import jax
import jax.numpy as jnp
from jax import lax
from jax.experimental import pallas as pl
from jax.experimental.pallas import tpu as pltpu
from jax.experimental.pallas import tpu_sc as plsc

N = 10000
E = 320000
D_NODE = 128
D_EDGE = 16
H = 4

NUM_CORES = 2
NUM_SUBCORES = 16
NUM_TILES = NUM_CORES * NUM_SUBCORES
EDGES_PER_TILE = E // NUM_TILES
HALF_N = N // 2

_sc_mesh = plsc.VectorSubcoreMesh(core_axis_name="c", subcore_axis_name="s")
_sc_params = pltpu.CompilerParams(use_tc_tiling_on_sc=False,
                                  needs_layout_passes=False)

_INV_SQRT2 = 0.7071067811865476


def _gelu_exact(x):
    return x * 0.5 * (1.0 + lax.erf(x * _INV_SQRT2))


@functools.partial(
    pl.kernel,
    out_type=(
        jax.ShapeDtypeStruct((H, N), jnp.float32),
        jax.ShapeDtypeStruct((H, N), jnp.float32),
    ),
    mesh=_sc_mesh,
    compiler_params=_sc_params,
    scratch_types=[
        pltpu.VMEM((EDGES_PER_TILE,), jnp.int32),
        pltpu.VMEM((EDGES_PER_TILE,), jnp.float32),
        pltpu.VMEM_SHARED((H, N), jnp.float32),
    ],
)
def _sc_scatter(ei_hbm, attnt_hbm, zeros_hbm, out0_hbm, out1_hbm,
                idx_v, vals_v, acc_sh):
    c = lax.axis_index("c")
    s = lax.axis_index("s")
    wid = c * NUM_SUBCORES + s

    @pl.when(s < 2 * H)
    def _():
        h = s // 2
        sl = pl.ds((s % 2) * HALF_N, HALF_N)
        pltpu.sync_copy(zeros_hbm.at[h].at[sl], acc_sh.at[h].at[sl])

    base = wid * EDGES_PER_TILE
    pltpu.sync_copy(ei_hbm.at[1].at[pl.ds(base, EDGES_PER_TILE)], idx_v)

    plsc.subcore_barrier()

    for h in range(H):
        pltpu.sync_copy(attnt_hbm.at[h].at[pl.ds(base, EDGES_PER_TILE)],
                        vals_v)
        pltpu.sync_copy(vals_v, acc_sh.at[h].at[idx_v], add=True)

    plsc.subcore_barrier()

    @pl.when(s < 2 * H)
    def _():
        h = s // 2
        sl = pl.ds((s % 2) * HALF_N, HALF_N)

        @pl.when(c == 0)
        def _():
            pltpu.sync_copy(acc_sh.at[h].at[sl], out0_hbm.at[h].at[sl])

        @pl.when(c == 1)
        def _():
            pltpu.sync_copy(acc_sh.at[h].at[sl], out1_hbm.at[h].at[sl])


def _node_mlp_body(p0_ref, p1_ref, nf_ref, wn1a_ref, wn1bt_ref, be1t_ref,
                   wn2_ref, bn2_ref, out_ref):
    napt = p0_ref[...] + p1_ref[...]
    hm = jnp.max(napt, axis=1, keepdims=True) + 1e-10
    napnt = napt / hm
    h1t = lax.dot_general(wn1a_ref[...], nf_ref[...],
                          (((0,), (1,)), ((), ())))
    for hh in range(H):
        h1t = h1t + wn1bt_ref[:, hh:hh + 1] * napnt[hh:hh + 1, :]
    h1t = h1t + be1t_ref[...]
    h1t = _gelu_exact(h1t)
    logits = jnp.sum(h1t * wn2_ref[...], axis=0, keepdims=True) + bn2_ref[...]
    out_ref[...] = jax.nn.sigmoid(logits)[0]


_node_mlp = pl.pallas_call(
    _node_mlp_body,
    out_shape=jax.ShapeDtypeStruct((N,), jnp.float32),
)


_GATHER_ITERS = EDGES_PER_TILE // 16


@functools.partial(
    pl.kernel,
    out_type=(
        jax.ShapeDtypeStruct((E,), jnp.float32),
        jax.ShapeDtypeStruct((E,), jnp.float32),
    ),
    mesh=_sc_mesh,
    compiler_params=_sc_params,
    scratch_types=[
        pltpu.VMEM((N,), jnp.float32),
        pltpu.VMEM((EDGES_PER_TILE,), jnp.int32),
        pltpu.VMEM((EDGES_PER_TILE,), jnp.int32),
        pltpu.VMEM((EDGES_PER_TILE,), jnp.float32),
        pltpu.VMEM((EDGES_PER_TILE,), jnp.float32),
    ],
)
def _sc_gather(ei_hbm, gates_hbm, outs_hbm, outt_hbm,
               gates_v, sidx_v, tidx_v, souts_v, soutt_v):
    c = lax.axis_index("c")
    s = lax.axis_index("s")
    wid = c * NUM_SUBCORES + s
    base = wid * EDGES_PER_TILE

    pltpu.sync_copy(gates_hbm, gates_v)
    pltpu.sync_copy(ei_hbm.at[0].at[pl.ds(base, EDGES_PER_TILE)], sidx_v)
    pltpu.sync_copy(ei_hbm.at[1].at[pl.ds(base, EDGES_PER_TILE)], tidx_v)

    def body(i, carry):
        sl = pl.ds(i * 16, 16)
        souts_v[sl] = plsc.load_gather(gates_v, [sidx_v[sl]])
        soutt_v[sl] = plsc.load_gather(gates_v, [tidx_v[sl]])
        return carry

    lax.fori_loop(0, _GATHER_ITERS, body, 0)

    pltpu.sync_copy(souts_v, outs_hbm.at[pl.ds(base, EDGES_PER_TILE)])
    pltpu.sync_copy(soutt_v, outt_hbm.at[pl.ds(base, EDGES_PER_TILE)])


EDGE_BLOCK = 6400
EDGE_GRID = E // EDGE_BLOCK


def _edge_mlp_body(eft_ref, attnt_ref, sg_ref, tg_ref, we1a_ref, we1bt_ref,
                   we1ct_ref, be1t_ref, we2_ref, be2_ref, out_ref):
    i = pl.program_id(0)
    esl = pl.ds(i * EDGE_BLOCK, EDGE_BLOCK)
    ht = lax.dot_general(we1a_ref[...], eft_ref[...],
                         (((0,), (0,)), ((), ())))
    attnt = attnt_ref[...]
    for hh in range(H):
        ht = ht + we1bt_ref[:, hh:hh + 1] * attnt[hh:hh + 1, :]
    ht = ht + we1ct_ref[:, 0:1] * sg_ref[esl]
    ht = ht + we1ct_ref[:, 1:2] * tg_ref[esl]
    ht = ht + be1t_ref[...]
    ht = _gelu_exact(ht)
    logits = jnp.sum(ht * we2_ref[...], axis=0, keepdims=True) + be2_ref[...]
    out_ref[esl] = jax.nn.sigmoid(logits)[0]


_edge_mlp = pl.pallas_call(
    _edge_mlp_body,
    grid=(EDGE_GRID,),
    in_specs=[
        pl.BlockSpec((D_EDGE, EDGE_BLOCK), lambda i: (0, i)),
        pl.BlockSpec((H, EDGE_BLOCK), lambda i: (0, i)),
        pl.BlockSpec((E,), lambda i: (0,)),
        pl.BlockSpec((E,), lambda i: (0,)),
        pl.BlockSpec((D_EDGE, D_EDGE), lambda i: (0, 0)),
        pl.BlockSpec((D_EDGE, H), lambda i: (0, 0)),
        pl.BlockSpec((D_EDGE, 2), lambda i: (0, 0)),
        pl.BlockSpec((D_EDGE, 1), lambda i: (0, 0)),
        pl.BlockSpec((D_EDGE, 1), lambda i: (0, 0)),
        pl.BlockSpec((1, 1), lambda i: (0, 0)),
    ],
    out_specs=pl.BlockSpec((E,), lambda i: (0,)),
    out_shape=jax.ShapeDtypeStruct((E,), jnp.float32),
)


def kernel(node_features, edge_features, edge_index, node_attn_weights,
           edge_attn_weights, Wn1, bn1, Wn2, bn2, We1, be1, We2, be2):
    attn_t = node_attn_weights.T

    zeros = jnp.zeros((H, N), jnp.float32)
    p0, p1 = _sc_scatter(edge_index, attn_t, zeros)

    node_gates = _node_mlp(
        p0, p1, node_features,
        Wn1[:D_NODE], Wn1[D_NODE:].T,
        bn1.reshape(-1, 1), Wn2, bn2.reshape(1, 1),
    )

    src_g, tgt_g = _sc_gather(edge_index, node_gates)

    edge_gates = _edge_mlp(
        edge_features.T, attn_t, src_g, tgt_g,
        We1[:D_EDGE], We1[D_EDGE:D_EDGE + H].T, We1[D_EDGE + H:].T,
        be1.reshape(-1, 1), We2, be2.reshape(1, 1),
    )

    return (node_gates, edge_gates)


# speedup vs baseline: 39.4964x; 1.0181x over previous
import functools

import jax
import jax.numpy as jnp
from jax import lax
from jax.experimental import pallas as pl
from jax.experimental.pallas import tpu as pltpu
from jax.experimental.pallas import tpu_sc as plsc

N = 10000
E = 320000
D_NODE = 128
D_EDGE = 16
H = 4

NUM_CORES = 2
NUM_SUBCORES = 16
NUM_TILES = NUM_CORES * NUM_SUBCORES
EDGES_PER_TILE = E // NUM_TILES
HALF_N = N // 2

_sc_mesh = plsc.VectorSubcoreMesh(core_axis_name="c", subcore_axis_name="s")
_sc_params = pltpu.CompilerParams(use_tc_tiling_on_sc=False,
                                  needs_layout_passes=False)

_INV_SQRT2 = 0.7071067811865476


def _gelu_exact(x):
    return x * 0.5 * (1.0 + lax.erf(x * _INV_SQRT2))


@functools.partial(
    pl.kernel,
    out_type=(
        jax.ShapeDtypeStruct((H, N), jnp.float32),
        jax.ShapeDtypeStruct((H, N), jnp.float32),
    ),
    mesh=_sc_mesh,
    compiler_params=_sc_params,
    scratch_types=[
        pltpu.VMEM((EDGES_PER_TILE,), jnp.int32),
        pltpu.VMEM((EDGES_PER_TILE,), jnp.float32),
        pltpu.VMEM_SHARED((H, N), jnp.float32),
    ],
)
def _sc_scatter(ei_hbm, attnt_hbm, zeros_hbm, out0_hbm, out1_hbm,
                idx_v, vals_v, acc_sh):
    c = lax.axis_index("c")
    s = lax.axis_index("s")
    wid = c * NUM_SUBCORES + s

    @pl.when(s < 2 * H)
    def _():
        h = s // 2
        sl = pl.ds((s % 2) * HALF_N, HALF_N)
        pltpu.sync_copy(zeros_hbm.at[h].at[sl], acc_sh.at[h].at[sl])

    base = wid * EDGES_PER_TILE
    pltpu.sync_copy(ei_hbm.at[1].at[pl.ds(base, EDGES_PER_TILE)], idx_v)

    plsc.subcore_barrier()

    for h in range(H):
        pltpu.sync_copy(attnt_hbm.at[h].at[pl.ds(base, EDGES_PER_TILE)],
                        vals_v)
        pltpu.sync_copy(vals_v, acc_sh.at[h].at[idx_v], add=True)

    plsc.subcore_barrier()

    @pl.when(s < 2 * H)
    def _():
        h = s // 2
        sl = pl.ds((s % 2) * HALF_N, HALF_N)

        @pl.when(c == 0)
        def _():
            pltpu.sync_copy(acc_sh.at[h].at[sl], out0_hbm.at[h].at[sl])

        @pl.when(c == 1)
        def _():
            pltpu.sync_copy(acc_sh.at[h].at[sl], out1_hbm.at[h].at[sl])


def _node_mlp_body(p0_ref, p1_ref, nf_ref, wn1a_ref, wn1bt_ref, be1t_ref,
                   wn2_ref, bn2_ref, out_ref):
    napt = p0_ref[...] + p1_ref[...]
    hm = jnp.max(napt, axis=1, keepdims=True) + 1e-10
    napnt = napt / hm
    h1t = lax.dot_general(wn1a_ref[...], nf_ref[...],
                          (((0,), (1,)), ((), ())))
    for hh in range(H):
        h1t = h1t + wn1bt_ref[:, hh:hh + 1] * napnt[hh:hh + 1, :]
    h1t = h1t + be1t_ref[...]
    h1t = _gelu_exact(h1t)
    logits = jnp.sum(h1t * wn2_ref[...], axis=0, keepdims=True) + bn2_ref[...]
    out_ref[...] = jax.nn.sigmoid(logits)[0]


_node_mlp = pl.pallas_call(
    _node_mlp_body,
    out_shape=jax.ShapeDtypeStruct((N,), jnp.float32),
)


_GATHER_ITERS = EDGES_PER_TILE // 16


@functools.partial(
    pl.kernel,
    out_type=(
        jax.ShapeDtypeStruct((E,), jnp.float32),
        jax.ShapeDtypeStruct((E,), jnp.float32),
    ),
    mesh=_sc_mesh,
    compiler_params=_sc_params,
    scratch_types=[
        pltpu.VMEM_SHARED((N,), jnp.float32),
        pltpu.VMEM((EDGES_PER_TILE,), jnp.int32),
        pltpu.VMEM((EDGES_PER_TILE,), jnp.int32),
        pltpu.VMEM((EDGES_PER_TILE,), jnp.float32),
        pltpu.VMEM((EDGES_PER_TILE,), jnp.float32),
    ],
)
def _sc_gather(ei_hbm, gates_hbm, outs_hbm, outt_hbm,
               gates_sh, sidx_v, tidx_v, souts_v, soutt_v):
    c = lax.axis_index("c")
    s = lax.axis_index("s")
    wid = c * NUM_SUBCORES + s
    base = wid * EDGES_PER_TILE

    @pl.when(s < 10)
    def _():
        sl = pl.ds(s * (N // 10), N // 10)
        pltpu.sync_copy(gates_hbm.at[sl], gates_sh.at[sl])

    pltpu.sync_copy(ei_hbm.at[0].at[pl.ds(base, EDGES_PER_TILE)], sidx_v)
    pltpu.sync_copy(ei_hbm.at[1].at[pl.ds(base, EDGES_PER_TILE)], tidx_v)

    plsc.subcore_barrier()

    pltpu.sync_copy(gates_sh.at[sidx_v], souts_v)
    pltpu.sync_copy(gates_sh.at[tidx_v], soutt_v)

    pltpu.sync_copy(souts_v, outs_hbm.at[pl.ds(base, EDGES_PER_TILE)])
    pltpu.sync_copy(soutt_v, outt_hbm.at[pl.ds(base, EDGES_PER_TILE)])


EDGE_BLOCK = 6400
EDGE_GRID = E // EDGE_BLOCK


def _edge_mlp_body(eft_ref, attnt_ref, sg_ref, tg_ref, we1a_ref, we1b_ref,
                   we1c_ref, be1t_ref, we2_ref, be2_ref, out_ref):
    i = pl.program_id(0)
    esl = pl.ds(i * EDGE_BLOCK, EDGE_BLOCK)
    cdims = (((0,), (0,)), ((), ()))
    ht = lax.dot_general(we1a_ref[...], eft_ref[...], cdims)
    ht = ht + lax.dot_general(we1b_ref[...], attnt_ref[...], cdims)
    sgtg = jnp.concatenate(
        [sg_ref[esl].reshape(1, EDGE_BLOCK), tg_ref[esl].reshape(1, EDGE_BLOCK)],
        axis=0)
    ht = ht + lax.dot_general(we1c_ref[...], sgtg, cdims)
    ht = ht + be1t_ref[...]
    ht = _gelu_exact(ht)
    logits = lax.dot_general(we2_ref[...], ht, cdims) + be2_ref[...]
    out_ref[esl] = jax.nn.sigmoid(logits)[0]


_edge_mlp = pl.pallas_call(
    _edge_mlp_body,
    grid=(EDGE_GRID,),
    in_specs=[
        pl.BlockSpec((D_EDGE, EDGE_BLOCK), lambda i: (0, i)),
        pl.BlockSpec((H, EDGE_BLOCK), lambda i: (0, i)),
        pl.BlockSpec((E,), lambda i: (0,)),
        pl.BlockSpec((E,), lambda i: (0,)),
        pl.BlockSpec((D_EDGE, D_EDGE), lambda i: (0, 0)),
        pl.BlockSpec((H, D_EDGE), lambda i: (0, 0)),
        pl.BlockSpec((2, D_EDGE), lambda i: (0, 0)),
        pl.BlockSpec((D_EDGE, 1), lambda i: (0, 0)),
        pl.BlockSpec((D_EDGE, 1), lambda i: (0, 0)),
        pl.BlockSpec((1, 1), lambda i: (0, 0)),
    ],
    out_specs=pl.BlockSpec((E,), lambda i: (0,)),
    out_shape=jax.ShapeDtypeStruct((E,), jnp.float32),
)


def kernel(node_features, edge_features, edge_index, node_attn_weights,
           edge_attn_weights, Wn1, bn1, Wn2, bn2, We1, be1, We2, be2):
    attn_t = node_attn_weights.T

    zeros = jnp.zeros((H, N), jnp.float32)
    p0, p1 = _sc_scatter(edge_index, attn_t, zeros)

    node_gates = _node_mlp(
        p0, p1, node_features,
        Wn1[:D_NODE], Wn1[D_NODE:].T,
        bn1.reshape(-1, 1), Wn2, bn2.reshape(1, 1),
    )

    src_g, tgt_g = _sc_gather(edge_index, node_gates)

    edge_gates = _edge_mlp(
        edge_features.T, attn_t, src_g, tgt_g,
        We1[:D_EDGE], We1[D_EDGE:D_EDGE + H], We1[D_EDGE + H:],
        be1.reshape(-1, 1), We2, be2.reshape(1, 1),
    )

    return (node_gates, edge_gates)


# speedup vs baseline: 39.5369x; 1.0010x over previous
import functools

import jax
import jax.numpy as jnp
from jax import lax
from jax.experimental import pallas as pl
from jax.experimental.pallas import tpu as pltpu
from jax.experimental.pallas import tpu_sc as plsc

N = 10000
E = 320000
D_NODE = 128
D_EDGE = 16
H = 4

NUM_CORES = 2
NUM_SUBCORES = 16
NUM_TILES = NUM_CORES * NUM_SUBCORES
EDGES_PER_TILE = E // NUM_TILES
HALF_N = N // 2

_sc_mesh = plsc.VectorSubcoreMesh(core_axis_name="c", subcore_axis_name="s")
_sc_params = pltpu.CompilerParams(use_tc_tiling_on_sc=False,
                                  needs_layout_passes=False)

_INV_SQRT2 = 0.7071067811865476


def _gelu_exact(x):
    return x * 0.5 * (1.0 + lax.erf(x * _INV_SQRT2))


@functools.partial(
    pl.kernel,
    out_type=(
        jax.ShapeDtypeStruct((H, N), jnp.float32),
        jax.ShapeDtypeStruct((H, N), jnp.float32),
    ),
    mesh=_sc_mesh,
    compiler_params=_sc_params,
    scratch_types=[
        pltpu.VMEM((EDGES_PER_TILE,), jnp.int32),
        pltpu.VMEM((EDGES_PER_TILE,), jnp.float32),
        pltpu.VMEM_SHARED((H, N), jnp.float32),
    ],
)
def _sc_scatter(ei_hbm, attnt_hbm, zeros_hbm, out0_hbm, out1_hbm,
                idx_v, vals_v, acc_sh):
    c = lax.axis_index("c")
    s = lax.axis_index("s")
    wid = c * NUM_SUBCORES + s

    @pl.when(s < 2 * H)
    def _():
        h = s // 2
        sl = pl.ds((s % 2) * HALF_N, HALF_N)
        pltpu.sync_copy(zeros_hbm.at[h].at[sl], acc_sh.at[h].at[sl])

    base = wid * EDGES_PER_TILE
    pltpu.sync_copy(ei_hbm.at[1].at[pl.ds(base, EDGES_PER_TILE)], idx_v)

    plsc.subcore_barrier()

    for h in range(H):
        pltpu.sync_copy(attnt_hbm.at[h].at[pl.ds(base, EDGES_PER_TILE)],
                        vals_v)
        pltpu.sync_copy(vals_v, acc_sh.at[h].at[idx_v], add=True)

    plsc.subcore_barrier()

    @pl.when(s < 2 * H)
    def _():
        h = s // 2
        sl = pl.ds((s % 2) * HALF_N, HALF_N)

        @pl.when(c == 0)
        def _():
            pltpu.sync_copy(acc_sh.at[h].at[sl], out0_hbm.at[h].at[sl])

        @pl.when(c == 1)
        def _():
            pltpu.sync_copy(acc_sh.at[h].at[sl], out1_hbm.at[h].at[sl])


def _node_mlp_body(p0_ref, p1_ref, nf_ref, wn1a_ref, wn1bt_ref, be1t_ref,
                   wn2_ref, bn2_ref, out_ref):
    napt = p0_ref[...] + p1_ref[...]
    hm = jnp.max(napt, axis=1, keepdims=True) + 1e-10
    napnt = napt / hm
    h1t = lax.dot_general(wn1a_ref[...], nf_ref[...],
                          (((0,), (1,)), ((), ())))
    for hh in range(H):
        h1t = h1t + wn1bt_ref[:, hh:hh + 1] * napnt[hh:hh + 1, :]
    h1t = h1t + be1t_ref[...]
    h1t = _gelu_exact(h1t)
    logits = jnp.sum(h1t * wn2_ref[...], axis=0, keepdims=True) + bn2_ref[...]
    out_ref[...] = jax.nn.sigmoid(logits)[0]


_node_mlp = pl.pallas_call(
    _node_mlp_body,
    out_shape=jax.ShapeDtypeStruct((N,), jnp.float32),
)


_GATHER_ITERS = EDGES_PER_TILE // 16


@functools.partial(
    pl.kernel,
    out_type=(
        jax.ShapeDtypeStruct((E,), jnp.float32),
        jax.ShapeDtypeStruct((E,), jnp.float32),
    ),
    mesh=_sc_mesh,
    compiler_params=_sc_params,
    scratch_types=[
        pltpu.VMEM_SHARED((N,), jnp.float32),
        pltpu.VMEM((EDGES_PER_TILE,), jnp.int32),
        pltpu.VMEM((EDGES_PER_TILE,), jnp.int32),
        pltpu.VMEM((EDGES_PER_TILE,), jnp.float32),
        pltpu.VMEM((EDGES_PER_TILE,), jnp.float32),
    ],
)
def _sc_gather(ei_hbm, gates_hbm, outs_hbm, outt_hbm,
               gates_sh, sidx_v, tidx_v, souts_v, soutt_v):
    c = lax.axis_index("c")
    s = lax.axis_index("s")
    wid = c * NUM_SUBCORES + s
    base = wid * EDGES_PER_TILE

    @pl.when(s < 10)
    def _():
        sl = pl.ds(s * (N // 10), N // 10)
        pltpu.sync_copy(gates_hbm.at[sl], gates_sh.at[sl])

    pltpu.sync_copy(ei_hbm.at[0].at[pl.ds(base, EDGES_PER_TILE)], sidx_v)
    pltpu.sync_copy(ei_hbm.at[1].at[pl.ds(base, EDGES_PER_TILE)], tidx_v)

    plsc.subcore_barrier()

    pltpu.sync_copy(gates_sh.at[sidx_v], souts_v)
    pltpu.sync_copy(gates_sh.at[tidx_v], soutt_v)

    pltpu.sync_copy(souts_v, outs_hbm.at[pl.ds(base, EDGES_PER_TILE)])
    pltpu.sync_copy(soutt_v, outt_hbm.at[pl.ds(base, EDGES_PER_TILE)])


EDGE_BLOCK = 6400
EDGE_GRID = E // EDGE_BLOCK


def _edge_mlp_body(eft_ref, attnt_ref, sg_ref, tg_ref, we1a_ref, we1b_ref,
                   we1c_ref, be1t_ref, we2_ref, be2_ref, out_ref):
    i = pl.program_id(0)
    esl = pl.ds(i * EDGE_BLOCK, EDGE_BLOCK)
    cdims = (((0,), (0,)), ((), ()))
    ht = lax.dot_general(we1a_ref[...], eft_ref[...], cdims)
    ht = ht + lax.dot_general(we1b_ref[...], attnt_ref[...], cdims)
    sgtg = jnp.concatenate(
        [sg_ref[esl].reshape(1, EDGE_BLOCK), tg_ref[esl].reshape(1, EDGE_BLOCK)],
        axis=0)
    ht = ht + lax.dot_general(we1c_ref[...], sgtg, cdims)
    ht = ht + be1t_ref[...]
    ht = _gelu_exact(ht)
    logits = lax.dot_general(we2_ref[...], ht, cdims) + be2_ref[...]
    out_ref[esl] = jax.nn.sigmoid(logits)[0]


_edge_mlp = pl.pallas_call(
    _edge_mlp_body,
    grid=(EDGE_GRID,),
    in_specs=[
        pl.BlockSpec((D_EDGE, EDGE_BLOCK), lambda i: (0, i)),
        pl.BlockSpec((H, EDGE_BLOCK), lambda i: (0, i)),
        pl.BlockSpec((E,), lambda i: (0,)),
        pl.BlockSpec((E,), lambda i: (0,)),
        pl.BlockSpec((D_EDGE, D_EDGE), lambda i: (0, 0)),
        pl.BlockSpec((H, D_EDGE), lambda i: (0, 0)),
        pl.BlockSpec((2, D_EDGE), lambda i: (0, 0)),
        pl.BlockSpec((D_EDGE, 1), lambda i: (0, 0)),
        pl.BlockSpec((D_EDGE, 1), lambda i: (0, 0)),
        pl.BlockSpec((1, 1), lambda i: (0, 0)),
    ],
    out_specs=pl.BlockSpec((E,), lambda i: (0,)),
    out_shape=jax.ShapeDtypeStruct((E,), jnp.float32),
    compiler_params=pltpu.CompilerParams(
        dimension_semantics=("parallel",)),
)


def kernel(node_features, edge_features, edge_index, node_attn_weights,
           edge_attn_weights, Wn1, bn1, Wn2, bn2, We1, be1, We2, be2):
    attn_t = node_attn_weights.T

    zeros = jnp.zeros((H, N), jnp.float32)
    p0, p1 = _sc_scatter(edge_index, attn_t, zeros)

    node_gates = _node_mlp(
        p0, p1, node_features,
        Wn1[:D_NODE], Wn1[D_NODE:].T,
        bn1.reshape(-1, 1), Wn2, bn2.reshape(1, 1),
    )

    src_g, tgt_g = _sc_gather(edge_index, node_gates)

    edge_gates = _edge_mlp(
        edge_features.T, attn_t, src_g, tgt_g,
        We1[:D_EDGE], We1[D_EDGE:D_EDGE + H], We1[D_EDGE + H:],
        be1.reshape(-1, 1), We2, be2.reshape(1, 1),
    )

    return (node_gates, edge_gates)


# speedup vs baseline: 46.0025x; 1.1635x over previous
import functools

import jax
import jax.numpy as jnp
from jax import lax
from jax.experimental import pallas as pl
from jax.experimental.pallas import tpu as pltpu
from jax.experimental.pallas import tpu_sc as plsc

N = 10000
E = 320000
D_NODE = 128
D_EDGE = 16
H = 4

NUM_CORES = 2
NUM_SUBCORES = 16
NUM_TILES = NUM_CORES * NUM_SUBCORES
EDGES_PER_TILE = E // NUM_TILES
HALF_N = N // 2

_sc_mesh = plsc.VectorSubcoreMesh(core_axis_name="c", subcore_axis_name="s")
_sc_params = pltpu.CompilerParams(use_tc_tiling_on_sc=False,
                                  needs_layout_passes=False)

_INV_SQRT2 = 0.7071067811865476


def _gelu_exact(x):
    return x * 0.5 * (1.0 + lax.erf(x * _INV_SQRT2))


@functools.partial(
    pl.kernel,
    out_type=(
        jax.ShapeDtypeStruct((H, N), jnp.float32),
        jax.ShapeDtypeStruct((H, N), jnp.float32),
    ),
    mesh=_sc_mesh,
    compiler_params=_sc_params,
    scratch_types=[
        pltpu.VMEM((EDGES_PER_TILE,), jnp.int32),
        pltpu.VMEM((EDGES_PER_TILE,), jnp.float32),
        pltpu.VMEM_SHARED((H, N), jnp.float32),
    ],
)
def _sc_scatter(ei_hbm, attnt_hbm, zeros_hbm, out0_hbm, out1_hbm,
                idx_v, vals_v, acc_sh):
    c = lax.axis_index("c")
    s = lax.axis_index("s")
    wid = c * NUM_SUBCORES + s

    @pl.when(s < 2 * H)
    def _():
        h = s // 2
        sl = pl.ds((s % 2) * HALF_N, HALF_N)
        pltpu.sync_copy(zeros_hbm.at[h].at[sl], acc_sh.at[h].at[sl])

    base = wid * EDGES_PER_TILE
    pltpu.sync_copy(ei_hbm.at[1].at[pl.ds(base, EDGES_PER_TILE)], idx_v)

    plsc.subcore_barrier()

    for h in range(H):
        pltpu.sync_copy(attnt_hbm.at[h].at[pl.ds(base, EDGES_PER_TILE)],
                        vals_v)
        pltpu.sync_copy(vals_v, acc_sh.at[h].at[idx_v], add=True)

    plsc.subcore_barrier()

    @pl.when(s < 2 * H)
    def _():
        h = s // 2
        sl = pl.ds((s % 2) * HALF_N, HALF_N)

        @pl.when(c == 0)
        def _():
            pltpu.sync_copy(acc_sh.at[h].at[sl], out0_hbm.at[h].at[sl])

        @pl.when(c == 1)
        def _():
            pltpu.sync_copy(acc_sh.at[h].at[sl], out1_hbm.at[h].at[sl])


def _node_mlp_body(p0_ref, p1_ref, nf_ref, wn1a_ref, wn1bt_ref, be1t_ref,
                   wn2_ref, bn2_ref, out_ref):
    napt = p0_ref[...] + p1_ref[...]
    hm = jnp.max(napt, axis=1, keepdims=True) + 1e-10
    napnt = napt / hm
    h1t = lax.dot_general(wn1a_ref[...], nf_ref[...],
                          (((0,), (1,)), ((), ())))
    for hh in range(H):
        h1t = h1t + wn1bt_ref[:, hh:hh + 1] * napnt[hh:hh + 1, :]
    h1t = h1t + be1t_ref[...]
    h1t = _gelu_exact(h1t)
    logits = jnp.sum(h1t * wn2_ref[...], axis=0, keepdims=True) + bn2_ref[...]
    out_ref[...] = jax.nn.sigmoid(logits)[0]


_node_mlp = pl.pallas_call(
    _node_mlp_body,
    out_shape=jax.ShapeDtypeStruct((N,), jnp.float32),
)


_GATHER_ITERS = EDGES_PER_TILE // 16


@functools.partial(
    pl.kernel,
    out_type=(
        jax.ShapeDtypeStruct((E,), jnp.float32),
        jax.ShapeDtypeStruct((E,), jnp.float32),
    ),
    mesh=_sc_mesh,
    compiler_params=_sc_params,
    scratch_types=[
        pltpu.VMEM_SHARED((N,), jnp.float32),
        pltpu.VMEM((EDGES_PER_TILE,), jnp.int32),
        pltpu.VMEM((EDGES_PER_TILE,), jnp.int32),
        pltpu.VMEM((EDGES_PER_TILE,), jnp.float32),
        pltpu.VMEM((EDGES_PER_TILE,), jnp.float32),
    ],
)
def _sc_gather(ei_hbm, gates_hbm, outs_hbm, outt_hbm,
               gates_sh, sidx_v, tidx_v, souts_v, soutt_v):
    c = lax.axis_index("c")
    s = lax.axis_index("s")
    wid = c * NUM_SUBCORES + s
    base = wid * EDGES_PER_TILE

    @pl.when(s < 10)
    def _():
        sl = pl.ds(s * (N // 10), N // 10)
        pltpu.sync_copy(gates_hbm.at[sl], gates_sh.at[sl])

    pltpu.sync_copy(ei_hbm.at[0].at[pl.ds(base, EDGES_PER_TILE)], sidx_v)
    pltpu.sync_copy(ei_hbm.at[1].at[pl.ds(base, EDGES_PER_TILE)], tidx_v)

    plsc.subcore_barrier()

    pltpu.sync_copy(gates_sh.at[sidx_v], souts_v)
    pltpu.sync_copy(gates_sh.at[tidx_v], soutt_v)

    pltpu.sync_copy(souts_v, outs_hbm.at[pl.ds(base, EDGES_PER_TILE)])
    pltpu.sync_copy(soutt_v, outt_hbm.at[pl.ds(base, EDGES_PER_TILE)])


EDGE_BLOCK = 12800
EDGE_GRID = E // EDGE_BLOCK


def _edge_mlp_body(eft_ref, attnt_ref, sg_ref, tg_ref, we1a_ref, we1b_ref,
                   we1c_ref, be1t_ref, we2_ref, be2_ref, out_ref):
    i = pl.program_id(0)
    esl = pl.ds(i * EDGE_BLOCK, EDGE_BLOCK)
    cdims = (((0,), (0,)), ((), ()))
    ht = lax.dot_general(we1a_ref[...], eft_ref[...], cdims)
    ht = ht + lax.dot_general(we1b_ref[...], attnt_ref[...], cdims)
    sgtg = jnp.concatenate(
        [sg_ref[esl].reshape(1, EDGE_BLOCK), tg_ref[esl].reshape(1, EDGE_BLOCK)],
        axis=0)
    ht = ht + lax.dot_general(we1c_ref[...], sgtg, cdims)
    ht = ht + be1t_ref[...]
    ht = _gelu_exact(ht)
    logits = lax.dot_general(we2_ref[...], ht, cdims) + be2_ref[...]
    out_ref[esl] = jax.nn.sigmoid(logits)[0]


_edge_mlp = pl.pallas_call(
    _edge_mlp_body,
    grid=(EDGE_GRID,),
    in_specs=[
        pl.BlockSpec((D_EDGE, EDGE_BLOCK), lambda i: (0, i)),
        pl.BlockSpec((H, EDGE_BLOCK), lambda i: (0, i)),
        pl.BlockSpec((E,), lambda i: (0,)),
        pl.BlockSpec((E,), lambda i: (0,)),
        pl.BlockSpec((D_EDGE, D_EDGE), lambda i: (0, 0)),
        pl.BlockSpec((H, D_EDGE), lambda i: (0, 0)),
        pl.BlockSpec((2, D_EDGE), lambda i: (0, 0)),
        pl.BlockSpec((D_EDGE, 1), lambda i: (0, 0)),
        pl.BlockSpec((D_EDGE, 1), lambda i: (0, 0)),
        pl.BlockSpec((1, 1), lambda i: (0, 0)),
    ],
    out_specs=pl.BlockSpec((E,), lambda i: (0,)),
    out_shape=jax.ShapeDtypeStruct((E,), jnp.float32),
    compiler_params=pltpu.CompilerParams(
        dimension_semantics=("parallel",)),
)


def kernel(node_features, edge_features, edge_index, node_attn_weights,
           edge_attn_weights, Wn1, bn1, Wn2, bn2, We1, be1, We2, be2):
    attn_t = node_attn_weights.T

    zeros = jnp.zeros((H, N), jnp.float32)
    p0, p1 = _sc_scatter(edge_index, attn_t, zeros)

    node_gates = _node_mlp(
        p0, p1, node_features,
        Wn1[:D_NODE], Wn1[D_NODE:].T,
        bn1.reshape(-1, 1), Wn2, bn2.reshape(1, 1),
    )

    src_g, tgt_g = _sc_gather(edge_index, node_gates)

    edge_gates = _edge_mlp(
        edge_features.T, attn_t, src_g, tgt_g,
        We1[:D_EDGE], We1[D_EDGE:D_EDGE + H], We1[D_EDGE + H:],
        be1.reshape(-1, 1), We2, be2.reshape(1, 1),
    )

    return (node_gates, edge_gates)


# speedup vs baseline: 50.4438x; 1.0965x over previous
import functools

import jax
import jax.numpy as jnp
from jax import lax
from jax.experimental import pallas as pl
from jax.experimental.pallas import tpu as pltpu
from jax.experimental.pallas import tpu_sc as plsc

N = 10000
E = 320000
D_NODE = 128
D_EDGE = 16
H = 4

NUM_CORES = 2
NUM_SUBCORES = 16
NUM_TILES = NUM_CORES * NUM_SUBCORES
EDGES_PER_TILE = E // NUM_TILES
HALF_N = N // 2

_sc_mesh = plsc.VectorSubcoreMesh(core_axis_name="c", subcore_axis_name="s")
_sc_params = pltpu.CompilerParams(use_tc_tiling_on_sc=False,
                                  needs_layout_passes=False)

_INV_SQRT2 = 0.7071067811865476


def _gelu_exact(x):
    return x * 0.5 * (1.0 + lax.erf(x * _INV_SQRT2))


@functools.partial(
    pl.kernel,
    out_type=(
        jax.ShapeDtypeStruct((H, N), jnp.float32),
        jax.ShapeDtypeStruct((H, N), jnp.float32),
    ),
    mesh=_sc_mesh,
    compiler_params=_sc_params,
    scratch_types=[
        pltpu.VMEM((EDGES_PER_TILE,), jnp.int32),
        pltpu.VMEM((EDGES_PER_TILE,), jnp.float32),
        pltpu.VMEM_SHARED((H, N), jnp.float32),
    ],
)
def _sc_scatter(ei_hbm, attnt_hbm, zeros_hbm, out0_hbm, out1_hbm,
                idx_v, vals_v, acc_sh):
    c = lax.axis_index("c")
    s = lax.axis_index("s")
    wid = c * NUM_SUBCORES + s

    @pl.when(s < 2 * H)
    def _():
        h = s // 2
        sl = pl.ds((s % 2) * HALF_N, HALF_N)
        pltpu.sync_copy(zeros_hbm.at[h].at[sl], acc_sh.at[h].at[sl])

    base = wid * EDGES_PER_TILE
    pltpu.sync_copy(ei_hbm.at[1].at[pl.ds(base, EDGES_PER_TILE)], idx_v)

    plsc.subcore_barrier()

    for h in range(H):
        pltpu.sync_copy(attnt_hbm.at[h].at[pl.ds(base, EDGES_PER_TILE)],
                        vals_v)
        pltpu.sync_copy(vals_v, acc_sh.at[h].at[idx_v], add=True)

    plsc.subcore_barrier()

    @pl.when(s < 2 * H)
    def _():
        h = s // 2
        sl = pl.ds((s % 2) * HALF_N, HALF_N)

        @pl.when(c == 0)
        def _():
            pltpu.sync_copy(acc_sh.at[h].at[sl], out0_hbm.at[h].at[sl])

        @pl.when(c == 1)
        def _():
            pltpu.sync_copy(acc_sh.at[h].at[sl], out1_hbm.at[h].at[sl])


def _node_mlp_body(p0_ref, p1_ref, nf_ref, wn1a_ref, wn1bt_ref, be1t_ref,
                   wn2_ref, bn2_ref, out_ref):
    napt = p0_ref[...] + p1_ref[...]
    hm = jnp.max(napt, axis=1, keepdims=True) + 1e-10
    napnt = napt / hm
    h1t = lax.dot_general(wn1a_ref[...], nf_ref[...],
                          (((0,), (1,)), ((), ())))
    for hh in range(H):
        h1t = h1t + wn1bt_ref[:, hh:hh + 1] * napnt[hh:hh + 1, :]
    h1t = h1t + be1t_ref[...]
    h1t = _gelu_exact(h1t)
    logits = jnp.sum(h1t * wn2_ref[...], axis=0, keepdims=True) + bn2_ref[...]
    out_ref[...] = jax.nn.sigmoid(logits)[0]


_node_mlp = pl.pallas_call(
    _node_mlp_body,
    out_shape=jax.ShapeDtypeStruct((N,), jnp.float32),
)


_GATHER_ITERS = EDGES_PER_TILE // 16


@functools.partial(
    pl.kernel,
    out_type=(
        jax.ShapeDtypeStruct((E,), jnp.float32),
        jax.ShapeDtypeStruct((E,), jnp.float32),
    ),
    mesh=_sc_mesh,
    compiler_params=_sc_params,
    scratch_types=[
        pltpu.VMEM_SHARED((N,), jnp.float32),
        pltpu.VMEM((EDGES_PER_TILE,), jnp.int32),
        pltpu.VMEM((EDGES_PER_TILE,), jnp.int32),
        pltpu.VMEM((EDGES_PER_TILE,), jnp.float32),
        pltpu.VMEM((EDGES_PER_TILE,), jnp.float32),
    ],
)
def _sc_gather(ei_hbm, gates_hbm, outs_hbm, outt_hbm,
               gates_sh, sidx_v, tidx_v, souts_v, soutt_v):
    c = lax.axis_index("c")
    s = lax.axis_index("s")
    wid = c * NUM_SUBCORES + s
    base = wid * EDGES_PER_TILE

    @pl.when(s < 10)
    def _():
        sl = pl.ds(s * (N // 10), N // 10)
        pltpu.sync_copy(gates_hbm.at[sl], gates_sh.at[sl])

    pltpu.sync_copy(ei_hbm.at[0].at[pl.ds(base, EDGES_PER_TILE)], sidx_v)
    pltpu.sync_copy(ei_hbm.at[1].at[pl.ds(base, EDGES_PER_TILE)], tidx_v)

    plsc.subcore_barrier()

    pltpu.sync_copy(gates_sh.at[sidx_v], souts_v)
    pltpu.sync_copy(gates_sh.at[tidx_v], soutt_v)

    pltpu.sync_copy(souts_v, outs_hbm.at[pl.ds(base, EDGES_PER_TILE)])
    pltpu.sync_copy(soutt_v, outt_hbm.at[pl.ds(base, EDGES_PER_TILE)])


EDGE_BLOCK = 32000
EDGE_GRID = E // EDGE_BLOCK


def _edge_mlp_body(eft_ref, attnt_ref, sg_ref, tg_ref, we1a_ref, we1b_ref,
                   we1c_ref, be1t_ref, we2_ref, be2_ref, out_ref):
    i = pl.program_id(0)
    esl = pl.ds(i * EDGE_BLOCK, EDGE_BLOCK)
    cdims = (((0,), (0,)), ((), ()))
    ht = lax.dot_general(we1a_ref[...], eft_ref[...], cdims)
    ht = ht + lax.dot_general(we1b_ref[...], attnt_ref[...], cdims)
    sgtg = jnp.concatenate(
        [sg_ref[esl].reshape(1, EDGE_BLOCK), tg_ref[esl].reshape(1, EDGE_BLOCK)],
        axis=0)
    ht = ht + lax.dot_general(we1c_ref[...], sgtg, cdims)
    ht = ht + be1t_ref[...]
    ht = _gelu_exact(ht)
    logits = lax.dot_general(we2_ref[...], ht, cdims) + be2_ref[...]
    out_ref[esl] = jax.nn.sigmoid(logits)[0]


_edge_mlp = pl.pallas_call(
    _edge_mlp_body,
    grid=(EDGE_GRID,),
    in_specs=[
        pl.BlockSpec((D_EDGE, EDGE_BLOCK), lambda i: (0, i)),
        pl.BlockSpec((H, EDGE_BLOCK), lambda i: (0, i)),
        pl.BlockSpec((E,), lambda i: (0,)),
        pl.BlockSpec((E,), lambda i: (0,)),
        pl.BlockSpec((D_EDGE, D_EDGE), lambda i: (0, 0)),
        pl.BlockSpec((H, D_EDGE), lambda i: (0, 0)),
        pl.BlockSpec((2, D_EDGE), lambda i: (0, 0)),
        pl.BlockSpec((D_EDGE, 1), lambda i: (0, 0)),
        pl.BlockSpec((D_EDGE, 1), lambda i: (0, 0)),
        pl.BlockSpec((1, 1), lambda i: (0, 0)),
    ],
    out_specs=pl.BlockSpec((E,), lambda i: (0,)),
    out_shape=jax.ShapeDtypeStruct((E,), jnp.float32),
    compiler_params=pltpu.CompilerParams(
        dimension_semantics=("parallel",)),
)


def kernel(node_features, edge_features, edge_index, node_attn_weights,
           edge_attn_weights, Wn1, bn1, Wn2, bn2, We1, be1, We2, be2):
    attn_t = node_attn_weights.T

    zeros = jnp.zeros((H, N), jnp.float32)
    p0, p1 = _sc_scatter(edge_index, attn_t, zeros)

    node_gates = _node_mlp(
        p0, p1, node_features,
        Wn1[:D_NODE], Wn1[D_NODE:].T,
        bn1.reshape(-1, 1), Wn2, bn2.reshape(1, 1),
    )

    src_g, tgt_g = _sc_gather(edge_index, node_gates)

    edge_gates = _edge_mlp(
        edge_features.T, attn_t, src_g, tgt_g,
        We1[:D_EDGE], We1[D_EDGE:D_EDGE + H], We1[D_EDGE + H:],
        be1.reshape(-1, 1), We2, be2.reshape(1, 1),
    )

    return (node_gates, edge_gates)


# speedup vs baseline: 50.5592x; 1.0023x over previous
import functools

import jax
import jax.numpy as jnp
from jax import lax
from jax.experimental import pallas as pl
from jax.experimental.pallas import tpu as pltpu
from jax.experimental.pallas import tpu_sc as plsc

N = 10000
E = 320000
D_NODE = 128
D_EDGE = 16
H = 4

NUM_CORES = 2
NUM_SUBCORES = 16
NUM_TILES = NUM_CORES * NUM_SUBCORES
EDGES_PER_TILE = E // NUM_TILES
HALF_N = N // 2

_sc_mesh = plsc.VectorSubcoreMesh(core_axis_name="c", subcore_axis_name="s")
_sc_params = pltpu.CompilerParams(use_tc_tiling_on_sc=False,
                                  needs_layout_passes=False)

_INV_SQRT2 = 0.7071067811865476


def _gelu_exact(x):
    return x * 0.5 * (1.0 + lax.erf(x * _INV_SQRT2))


@functools.partial(
    pl.kernel,
    out_type=(
        jax.ShapeDtypeStruct((H, N), jnp.float32),
        jax.ShapeDtypeStruct((H, N), jnp.float32),
    ),
    mesh=_sc_mesh,
    compiler_params=_sc_params,
    scratch_types=[
        pltpu.VMEM((EDGES_PER_TILE,), jnp.int32),
        pltpu.VMEM((EDGES_PER_TILE,), jnp.float32),
        pltpu.VMEM_SHARED((H, N), jnp.float32),
    ],
)
def _sc_scatter(ei_hbm, attnt_hbm, zeros_hbm, out0_hbm, out1_hbm,
                idx_v, vals_v, acc_sh):
    c = lax.axis_index("c")
    s = lax.axis_index("s")
    wid = c * NUM_SUBCORES + s

    @pl.when(s < 2 * H)
    def _():
        h = s // 2
        sl = pl.ds((s % 2) * HALF_N, HALF_N)
        pltpu.sync_copy(zeros_hbm.at[h].at[sl], acc_sh.at[h].at[sl])

    base = wid * EDGES_PER_TILE
    pltpu.sync_copy(ei_hbm.at[1].at[pl.ds(base, EDGES_PER_TILE)], idx_v)

    plsc.subcore_barrier()

    for h in range(H):
        pltpu.sync_copy(attnt_hbm.at[h].at[pl.ds(base, EDGES_PER_TILE)],
                        vals_v)
        pltpu.sync_copy(vals_v, acc_sh.at[h].at[idx_v], add=True)

    plsc.subcore_barrier()

    @pl.when(s < 2 * H)
    def _():
        h = s // 2
        sl = pl.ds((s % 2) * HALF_N, HALF_N)

        @pl.when(c == 0)
        def _():
            pltpu.sync_copy(acc_sh.at[h].at[sl], out0_hbm.at[h].at[sl])

        @pl.when(c == 1)
        def _():
            pltpu.sync_copy(acc_sh.at[h].at[sl], out1_hbm.at[h].at[sl])


def _node_mlp_body(p0_ref, p1_ref, nf_ref, wn1a_ref, wn1bt_ref, be1t_ref,
                   wn2_ref, bn2_ref, out_ref):
    napt = p0_ref[...] + p1_ref[...]
    hm = jnp.max(napt, axis=1, keepdims=True) + 1e-10
    napnt = napt / hm
    h1t = lax.dot_general(wn1a_ref[...], nf_ref[...],
                          (((0,), (1,)), ((), ())))
    for hh in range(H):
        h1t = h1t + wn1bt_ref[:, hh:hh + 1] * napnt[hh:hh + 1, :]
    h1t = h1t + be1t_ref[...]
    h1t = _gelu_exact(h1t)
    logits = jnp.sum(h1t * wn2_ref[...], axis=0, keepdims=True) + bn2_ref[...]
    out_ref[...] = jax.nn.sigmoid(logits)[0]


_node_mlp = pl.pallas_call(
    _node_mlp_body,
    out_shape=jax.ShapeDtypeStruct((N,), jnp.float32),
)


_GATHER_ITERS = EDGES_PER_TILE // 16


@functools.partial(
    pl.kernel,
    out_type=(
        jax.ShapeDtypeStruct((E,), jnp.float32),
        jax.ShapeDtypeStruct((E,), jnp.float32),
    ),
    mesh=_sc_mesh,
    compiler_params=_sc_params,
    scratch_types=[
        pltpu.VMEM_SHARED((N,), jnp.float32),
        pltpu.VMEM((EDGES_PER_TILE,), jnp.int32),
        pltpu.VMEM((EDGES_PER_TILE,), jnp.int32),
        pltpu.VMEM((EDGES_PER_TILE,), jnp.float32),
        pltpu.VMEM((EDGES_PER_TILE,), jnp.float32),
    ],
)
def _sc_gather(ei_hbm, gates_hbm, outs_hbm, outt_hbm,
               gates_sh, sidx_v, tidx_v, souts_v, soutt_v):
    c = lax.axis_index("c")
    s = lax.axis_index("s")
    wid = c * NUM_SUBCORES + s
    base = wid * EDGES_PER_TILE

    @pl.when(s < 10)
    def _():
        sl = pl.ds(s * (N // 10), N // 10)
        pltpu.sync_copy(gates_hbm.at[sl], gates_sh.at[sl])

    pltpu.sync_copy(ei_hbm.at[0].at[pl.ds(base, EDGES_PER_TILE)], sidx_v)
    pltpu.sync_copy(ei_hbm.at[1].at[pl.ds(base, EDGES_PER_TILE)], tidx_v)

    plsc.subcore_barrier()

    pltpu.sync_copy(gates_sh.at[sidx_v], souts_v)
    pltpu.sync_copy(gates_sh.at[tidx_v], soutt_v)

    pltpu.sync_copy(souts_v, outs_hbm.at[pl.ds(base, EDGES_PER_TILE)])
    pltpu.sync_copy(soutt_v, outt_hbm.at[pl.ds(base, EDGES_PER_TILE)])


EDGE_BLOCK = 64000
EDGE_GRID = E // EDGE_BLOCK


def _edge_mlp_body(eft_ref, attnt_ref, sg_ref, tg_ref, we1a_ref, we1b_ref,
                   we1c_ref, be1t_ref, we2_ref, be2_ref, out_ref):
    i = pl.program_id(0)
    esl = pl.ds(i * EDGE_BLOCK, EDGE_BLOCK)
    cdims = (((0,), (0,)), ((), ()))
    ht = lax.dot_general(we1a_ref[...], eft_ref[...], cdims)
    ht = ht + lax.dot_general(we1b_ref[...], attnt_ref[...], cdims)
    sgtg = jnp.concatenate(
        [sg_ref[esl].reshape(1, EDGE_BLOCK), tg_ref[esl].reshape(1, EDGE_BLOCK)],
        axis=0)
    ht = ht + lax.dot_general(we1c_ref[...], sgtg, cdims)
    ht = ht + be1t_ref[...]
    ht = _gelu_exact(ht)
    logits = lax.dot_general(we2_ref[...], ht, cdims) + be2_ref[...]
    out_ref[esl] = jax.nn.sigmoid(logits)[0]


_edge_mlp = pl.pallas_call(
    _edge_mlp_body,
    grid=(EDGE_GRID,),
    in_specs=[
        pl.BlockSpec((D_EDGE, EDGE_BLOCK), lambda i: (0, i)),
        pl.BlockSpec((H, EDGE_BLOCK), lambda i: (0, i)),
        pl.BlockSpec((E,), lambda i: (0,)),
        pl.BlockSpec((E,), lambda i: (0,)),
        pl.BlockSpec((D_EDGE, D_EDGE), lambda i: (0, 0)),
        pl.BlockSpec((H, D_EDGE), lambda i: (0, 0)),
        pl.BlockSpec((2, D_EDGE), lambda i: (0, 0)),
        pl.BlockSpec((D_EDGE, 1), lambda i: (0, 0)),
        pl.BlockSpec((D_EDGE, 1), lambda i: (0, 0)),
        pl.BlockSpec((1, 1), lambda i: (0, 0)),
    ],
    out_specs=pl.BlockSpec((E,), lambda i: (0,)),
    out_shape=jax.ShapeDtypeStruct((E,), jnp.float32),
    compiler_params=pltpu.CompilerParams(
        dimension_semantics=("parallel",)),
)


def kernel(node_features, edge_features, edge_index, node_attn_weights,
           edge_attn_weights, Wn1, bn1, Wn2, bn2, We1, be1, We2, be2):
    attn_t = node_attn_weights.T

    zeros = jnp.zeros((H, N), jnp.float32)
    p0, p1 = _sc_scatter(edge_index, attn_t, zeros)

    node_gates = _node_mlp(
        p0, p1, node_features,
        Wn1[:D_NODE], Wn1[D_NODE:].T,
        bn1.reshape(-1, 1), Wn2, bn2.reshape(1, 1),
    )

    src_g, tgt_g = _sc_gather(edge_index, node_gates)

    edge_gates = _edge_mlp(
        edge_features.T, attn_t, src_g, tgt_g,
        We1[:D_EDGE], We1[D_EDGE:D_EDGE + H], We1[D_EDGE + H:],
        be1.reshape(-1, 1), We2, be2.reshape(1, 1),
    )

    return (node_gates, edge_gates)
